# baseline (speedup 1.0000x reference)
import functools

import jax
import jax.numpy as jnp
from jax import lax
from jax.experimental import pallas as pl
from jax.experimental.pallas import tpu as pltpu
from jax.experimental.pallas import tpu_sc as plsc

VOCAB = 1010
REPR = 64
BATCH = 16384

NUM_CORES = 2
NUM_SUBCORES = 16
NUM_WORKERS = NUM_CORES * NUM_SUBCORES
BPW = BATCH // NUM_WORKERS
CH = 256
CGROUPS = CH // 16
DUNROLL = 16
DBLOCKS = REPR // DUNROLL


def _rsqrt(x):
    i = plsc.bitcast(x, jnp.int32)
    y = plsc.bitcast(jnp.int32(0x5F3759DF) - (i >> 1), jnp.float32)
    for _ in range(3):
        y = y * (1.5 - 0.5 * x * y * y)
    return y


def _body(
    rept_hbm, idx_hbm, table_hbm, out_hbm,
    idx_v, pair_v, out_v, rows0, rows1, rep0, rep1,
    gsem0, gsem1, rsem0, rsem1,
):
    wid = lax.axis_index("s") * NUM_CORES + lax.axis_index("c")
    base = wid * BPW

    pltpu.sync_copy(idx_hbm.at[pl.ds(base, BPW)], idx_v)

    def halve(g, carry):
        pair_v[pl.ds(g * 16, 16)] = idx_v[pl.ds(g * 16, 16)] >> 1
        return carry

    lax.fori_loop(0, BPW // 16, halve, 0)

    def start(off, rows_v, rep_v, gsem, rsem):
        g = pltpu.async_copy(table_hbm.at[pair_v.at[pl.ds(off, CH)]], rows_v, gsem)
        r = pltpu.async_copy(rept_hbm.at[:, pl.ds(base + off, CH)], rep_v, rsem)
        return g, r

    iota16 = lax.iota(jnp.int32, 16)

    def compute(off, rows_v, rep_v):
        def group(g, carry):
            loc = g * 16 + iota16
            half = (idx_v[pl.ds(off + g * 16, 16)] & 1) << 6

            zero = jnp.zeros((16,), jnp.float32)
            dot, nc, nr = zero, zero, zero
            for d in range(REPR):
                col = (iota16 + d) & (REPR - 1)
                c = plsc.load_gather(rows_v, [loc, half + col])
                r = plsc.load_gather(rep_v, [col, loc])
                dot = dot + c * r
                nc = nc + c * c
                nr = nr + r * r
            inv = _rsqrt(jnp.maximum(nc, 1e-24)) * _rsqrt(jnp.maximum(nr, 1e-24))
            plsc.store_scatter(out_v, [off + loc], 1.0 - dot * inv)
            return carry

        lax.fori_loop(0, CGROUPS, group, 0)

    g0, r0 = start(0, rows0, rep0, gsem0, rsem0)
    g1, r1 = start(CH, rows1, rep1, gsem1, rsem1)
    g0.wait()
    r0.wait()
    compute(0, rows0, rep0)
    g1.wait()
    r1.wait()
    compute(CH, rows1, rep1)

    pltpu.sync_copy(out_v, out_hbm.at[pl.ds(base, BPW)])


@jax.jit
def kernel(rep, expr, emb_weight):
    table2 = emb_weight.reshape(VOCAB // 2, 2 * REPR)
    rept = rep.T
    mesh = plsc.VectorSubcoreMesh(core_axis_name="c", subcore_axis_name="s")
    run = functools.partial(
        pl.kernel,
        out_type=jax.ShapeDtypeStruct((BATCH,), jnp.float32),
        mesh=mesh,
        scratch_types=[
            pltpu.VMEM((BPW,), jnp.int32),
            pltpu.VMEM((BPW,), jnp.int32),
            pltpu.VMEM((BPW,), jnp.float32),
            pltpu.VMEM((CH, 2 * REPR), jnp.float32),
            pltpu.VMEM((CH, 2 * REPR), jnp.float32),
            pltpu.VMEM((REPR, CH), jnp.float32),
            pltpu.VMEM((REPR, CH), jnp.float32),
            pltpu.SemaphoreType.DMA,
            pltpu.SemaphoreType.DMA,
            pltpu.SemaphoreType.DMA,
            pltpu.SemaphoreType.DMA,
        ],
        compiler_params=pltpu.CompilerParams(needs_layout_passes=False),
    )(_body)
    return run(rept, expr, table2)


# speedup vs baseline: 1.0307x; 1.0307x over previous
import functools

import jax
import jax.numpy as jnp
from jax import lax
from jax.experimental import pallas as pl
from jax.experimental.pallas import tpu as pltpu
from jax.experimental.pallas import tpu_sc as plsc

VOCAB = 1010
REPR = 64
BATCH = 16384

NUM_CORES = 2
NUM_SUBCORES = 16
NUM_WORKERS = NUM_CORES * NUM_SUBCORES
BPW = BATCH // NUM_WORKERS
CH = 256
CGROUPS = CH // 16
DUNROLL = 32
DBLOCKS = REPR // DUNROLL


def _rsqrt(x):
    i = plsc.bitcast(x, jnp.int32)
    y = plsc.bitcast(jnp.int32(0x5F3759DF) - (i >> 1), jnp.float32)
    for _ in range(2):
        y = y * (1.5 - 0.5 * x * y * y)
    return y


def _body(
    rept_hbm, idx_hbm, table_hbm, out_hbm,
    idx_v, pair_v, out_v, rows0, rows1, rep0, rep1,
    gsem0, gsem1, rsem0, rsem1,
):
    wid = lax.axis_index("s") * NUM_CORES + lax.axis_index("c")
    base = wid * BPW

    r0 = pltpu.async_copy(rept_hbm.at[:, pl.ds(base, CH)], rep0, rsem0)
    r1 = pltpu.async_copy(rept_hbm.at[:, pl.ds(base + CH, CH)], rep1, rsem1)

    pltpu.sync_copy(idx_hbm.at[pl.ds(base, BPW)], idx_v)

    def halve(g, carry):
        pair_v[pl.ds(g * 16, 16)] = idx_v[pl.ds(g * 16, 16)] >> 1
        return carry

    lax.fori_loop(0, BPW // 16, halve, 0)

    iota16 = lax.iota(jnp.int32, 16)

    def compute(off, rows_v, rep_v):
        def group(g, carry):
            loc = g * 16 + iota16
            half = (idx_v[pl.ds(off + g * 16, 16)] & 1) << 6

            def dblock(db, acc):
                dot, nc, nr = acc
                cbase = iota16 + db * DUNROLL
                for u in range(DUNROLL):
                    col = (cbase + u) & (REPR - 1)
                    c = plsc.load_gather(rows_v, [loc, half + col])
                    r = plsc.load_gather(rep_v, [col, loc])
                    dot = dot + c * r
                    nc = nc + c * c
                    nr = nr + r * r
                return dot, nc, nr

            zero = jnp.zeros((16,), jnp.float32)
            dot, nc, nr = lax.fori_loop(0, DBLOCKS, dblock, (zero, zero, zero))
            inv = _rsqrt(jnp.maximum(nc, 1e-24)) * _rsqrt(jnp.maximum(nr, 1e-24))
            plsc.store_scatter(out_v, [off + loc], 1.0 - dot * inv)
            return carry

        lax.fori_loop(0, CGROUPS, group, 0)

    g0 = pltpu.async_copy(table_hbm.at[pair_v.at[pl.ds(0, CH)]], rows0, gsem0)
    g1 = pltpu.async_copy(table_hbm.at[pair_v.at[pl.ds(CH, CH)]], rows1, gsem1)
    g0.wait()
    r0.wait()
    compute(0, rows0, rep0)
    g1.wait()
    r1.wait()
    compute(CH, rows1, rep1)

    pltpu.sync_copy(out_v, out_hbm.at[pl.ds(base, BPW)])


@jax.jit
def kernel(rep, expr, emb_weight):
    table2 = emb_weight.reshape(VOCAB // 2, 2 * REPR)
    rept = rep.T
    mesh = plsc.VectorSubcoreMesh(core_axis_name="c", subcore_axis_name="s")
    run = functools.partial(
        pl.kernel,
        out_type=jax.ShapeDtypeStruct((BATCH,), jnp.float32),
        mesh=mesh,
        scratch_types=[
            pltpu.VMEM((BPW,), jnp.int32),
            pltpu.VMEM((BPW,), jnp.int32),
            pltpu.VMEM((BPW,), jnp.float32),
            pltpu.VMEM((CH, 2 * REPR), jnp.float32),
            pltpu.VMEM((CH, 2 * REPR), jnp.float32),
            pltpu.VMEM((REPR, CH), jnp.float32),
            pltpu.VMEM((REPR, CH), jnp.float32),
            pltpu.SemaphoreType.DMA,
            pltpu.SemaphoreType.DMA,
            pltpu.SemaphoreType.DMA,
            pltpu.SemaphoreType.DMA,
        ],
        compiler_params=pltpu.CompilerParams(needs_layout_passes=False),
    )(_body)
    return run(rept, expr, table2)


# speedup vs baseline: 1.0944x; 1.0618x over previous
import functools

import jax
import jax.numpy as jnp
from jax import lax
from jax.experimental import pallas as pl
from jax.experimental.pallas import tpu as pltpu
from jax.experimental.pallas import tpu_sc as plsc

VOCAB = 1010
REPR = 64
BATCH = 16384

NUM_CORES = 2
NUM_SUBCORES = 16
NUM_WORKERS = NUM_CORES * NUM_SUBCORES
BPW = BATCH // NUM_WORKERS
CH = 256
CGROUPS = CH // 16
DUNROLL = 16
DBLOCKS = REPR // DUNROLL


def _rsqrt(x):
    i = plsc.bitcast(x, jnp.int32)
    y = plsc.bitcast(jnp.int32(0x5F3759DF) - (i >> 1), jnp.float32)
    for _ in range(2):
        y = y * (1.5 - 0.5 * x * y * y)
    return y


def _body(
    rept_hbm, idx_hbm, table_hbm, out_hbm,
    idx_v, pair_v, out_v, rows0, rows1, rep0, rep1,
    gsem0, gsem1, rsem0, rsem1,
):
    wid = lax.axis_index("s") * NUM_CORES + lax.axis_index("c")
    base = wid * BPW

    r0 = pltpu.async_copy(rept_hbm.at[:, pl.ds(base, CH)], rep0, rsem0)
    r1 = pltpu.async_copy(rept_hbm.at[:, pl.ds(base + CH, CH)], rep1, rsem1)

    pltpu.sync_copy(idx_hbm.at[pl.ds(base, BPW)], idx_v)

    def halve(g, carry):
        pair_v[pl.ds(g * 16, 16)] = idx_v[pl.ds(g * 16, 16)] >> 1
        return carry

    lax.fori_loop(0, BPW // 16, halve, 0)

    iota16 = lax.iota(jnp.int32, 16)

    def compute(off, rows_v, rep_v):
        def group(g, carry):
            loc = g * 16 + iota16
            half = (idx_v[pl.ds(off + g * 16, 16)] & 1) << 6

            def dblock(db, acc):
                dot, nc, nr = acc
                cbase = iota16 + db * DUNROLL
                for u in range(DUNROLL):
                    col = (cbase + u) & (REPR - 1)
                    c = plsc.load_gather(rows_v, [loc, half + col])
                    r = plsc.load_gather(rep_v, [col, loc])
                    dot = dot + c * r
                    nc = nc + c * c
                    nr = nr + r * r
                return dot, nc, nr

            zero = jnp.zeros((16,), jnp.float32)
            dot, nc, nr = lax.fori_loop(0, DBLOCKS, dblock, (zero, zero, zero))
            inv = _rsqrt(jnp.maximum(nc, 1e-24)) * _rsqrt(jnp.maximum(nr, 1e-24))
            plsc.store_scatter(out_v, [off + loc], 1.0 - dot * inv)
            return carry

        lax.fori_loop(0, CGROUPS, group, 0)

    g0 = pltpu.async_copy(table_hbm.at[pair_v.at[pl.ds(0, CH)]], rows0, gsem0)
    g1 = pltpu.async_copy(table_hbm.at[pair_v.at[pl.ds(CH, CH)]], rows1, gsem1)
    g0.wait()
    r0.wait()
    compute(0, rows0, rep0)
    g1.wait()
    r1.wait()
    compute(CH, rows1, rep1)

    pltpu.sync_copy(out_v, out_hbm.at[pl.ds(base, BPW)])


@jax.jit
def kernel(rep, expr, emb_weight):
    table2 = emb_weight.reshape(VOCAB // 2, 2 * REPR)
    rept = rep.T
    mesh = plsc.VectorSubcoreMesh(core_axis_name="c", subcore_axis_name="s")
    run = functools.partial(
        pl.kernel,
        out_type=jax.ShapeDtypeStruct((BATCH,), jnp.float32),
        mesh=mesh,
        scratch_types=[
            pltpu.VMEM((BPW,), jnp.int32),
            pltpu.VMEM((BPW,), jnp.int32),
            pltpu.VMEM((BPW,), jnp.float32),
            pltpu.VMEM((CH, 2 * REPR), jnp.float32),
            pltpu.VMEM((CH, 2 * REPR), jnp.float32),
            pltpu.VMEM((REPR, CH), jnp.float32),
            pltpu.VMEM((REPR, CH), jnp.float32),
            pltpu.SemaphoreType.DMA,
            pltpu.SemaphoreType.DMA,
            pltpu.SemaphoreType.DMA,
            pltpu.SemaphoreType.DMA,
        ],
        compiler_params=pltpu.CompilerParams(needs_layout_passes=False),
    )(_body)
    return run(rept, expr, table2)
